# baseline (speedup 1.0000x reference)
import functools

import jax
import jax.numpy as jnp
from jax import lax
from jax.experimental import pallas as pl
from jax.experimental.pallas import tpu as pltpu
from jax.experimental.pallas import tpu_sc as plsc

try:
    _info = plsc.get_sparse_core_info()
    _NC, _NS = int(_info.num_cores), int(_info.num_subcores)
except Exception:
    _NC, _NS = 2, 16

_B = 80


def _edge_den(src_e, dst_e, tsrc, tdst, z8f):
    N = tsrc.shape[0]
    E = src_e.shape[0]
    NW = _NC * _NS
    ept = E // NW
    nchunks = ept // _B

    @functools.partial(
        pl.kernel,
        out_type=(
            jax.ShapeDtypeStruct((E, 8), jnp.float32),
            jax.ShapeDtypeStruct((NW, N * 8), jnp.float32),
        ),
        mesh=plsc.VectorSubcoreMesh(core_axis_name="c", subcore_axis_name="s"),
        compiler_params=pltpu.CompilerParams(
            needs_layout_passes=False, use_tc_tiling_on_sc=False),
        scratch_types=[
            pltpu.VMEM((4, _B), jnp.int32),
            pltpu.VMEM((4, _B), jnp.int32),
            pltpu.VMEM((2 * _B, 8), jnp.float32),
            pltpu.VMEM((2 * _B, 8), jnp.float32),
            pltpu.VMEM((2 * _B, 8), jnp.float32),
            pltpu.VMEM((N * 8,), jnp.float32),
            pltpu.SemaphoreType.DMA,
            pltpu.SemaphoreType.DMA,
            pltpu.SemaphoreType.DMA,
            pltpu.SemaphoreType.DMA,
            pltpu.SemaphoreType.DMA,
            pltpu.SemaphoreType.DMA,
            pltpu.SemaphoreType.DMA,
            pltpu.SemaphoreType.DMA,
        ],
    )
    def k(srce_h, dste_h, tsrc_h, tdst_h, z8f_h, p_h, den_h,
          srcr, dstr, as_v, ad_v, p_v, den_t,
          si0, si1, si2, si3, sg0, sg1, ss0, ss1):
        si = [si0, si1, si2, si3]
        sg = [sg0, sg1]
        ss = [ss0, ss1]
        cid = lax.axis_index("c")
        sid = lax.axis_index("s")
        tid = cid * _NS + sid
        pltpu.sync_copy(z8f_h, den_t)

        iot = lax.iota(jnp.int32, 16)
        icol = lax.rem(iot, 8)
        irow = lax.div(iot, 8)
        m_lo = iot < 8
        m_hi = iot >= 8

        def idx_issue(c, q):
            base = tid * ept + c * _B
            pltpu.async_copy(srce_h.at[pl.ds(base, _B)], srcr.at[q], si[q])
            pltpu.async_copy(dste_h.at[pl.ds(base, _B)], dstr.at[q], si[q])

        def idx_wait(c, q):
            base = tid * ept + c * _B
            pltpu.make_async_copy(
                srce_h.at[pl.ds(base, _B)], srcr.at[q], si[q]).wait()
            pltpu.make_async_copy(
                dste_h.at[pl.ds(base, _B)], dstr.at[q], si[q]).wait()

        def gat_issue2(c, b, q):
            pltpu.async_copy(tsrc_h.at[srcr.at[q]],
                             as_v.at[pl.ds(b * _B, _B)], sg[b])
            pltpu.async_copy(tdst_h.at[dstr.at[q]],
                             ad_v.at[pl.ds(b * _B, _B)], sg[b])

        def gat_wait2(c, b, q):
            pltpu.make_async_copy(tsrc_h.at[srcr.at[q]],
                                  as_v.at[pl.ds(b * _B, _B)], sg[b]).wait()
            pltpu.make_async_copy(tdst_h.at[dstr.at[q]],
                                  ad_v.at[pl.ds(b * _B, _B)], sg[b]).wait()

        def compute(c, b, q):
            boff = b * _B
            sq = jnp.full((16,), q, jnp.int32)

            def vb(j, c2):
                ia = boff + 2 * j + irow
                s = (plsc.load_gather(as_v, [ia, icol])
                     + plsc.load_gather(ad_v, [ia, icol]))
                p = jnp.exp(jnp.maximum(s, 0.2 * s))
                plsc.store_scatter(p_v, [ia, icol], p)
                dstg = plsc.load_gather(dstr, [sq, 2 * j + irow])
                fidx = dstg * 8 + icol
                plsc.addupdate_scatter(den_t, [fidx], p, mask=m_lo)
                plsc.addupdate_scatter(den_t, [fidx], p, mask=m_hi)
                return c2

            lax.fori_loop(0, _B // 2, vb, 0)

        def st_issue2(c, b, q):
            base = tid * ept + c * _B
            pltpu.async_copy(p_v.at[pl.ds(b * _B, _B)],
                             p_h.at[pl.ds(base, _B)], ss[b])

        def st_wait2(c, b, q):
            base = tid * ept + c * _B
            pltpu.make_async_copy(p_v.at[pl.ds(b * _B, _B)],
                                  p_h.at[pl.ds(base, _B)], ss[b]).wait()

        def step(c, r, n1, n2, prev):
            if n1:
                idx_wait(c + 1, (r + 1) % 4)
            if n2:
                idx_issue(c + 2, (r + 2) % 4)
            gat_wait2(c, r % 2, r)
            if n1:
                gat_issue2(c + 1, (r + 1) % 2, (r + 1) % 4)
            compute(c, r % 2, r)
            if prev:
                st_wait2(c - 1, (r - 1) % 2, (r - 1) % 4)
            st_issue2(c, r % 2, r)

        idx_issue(0, 0)
        idx_wait(0, 0)
        gat_issue2(0, 0, 0)
        idx_issue(1, 1)
        step(0, 0, True, True, False)

        def quad(t, carry):
            for cc in range(4):
                step(4 * t + 1 + cc, (1 + cc) % 4, True, True, True)
            return carry

        lax.fori_loop(0, (nchunks - 5) // 4, quad, 0)
        for c in range(nchunks - 4, nchunks):
            step(c, c % 4, c + 1 < nchunks, c + 2 < nchunks, True)
        st_wait2(nchunks - 1, (nchunks - 1) % 2, (nchunks - 1) % 4)

        pltpu.sync_copy(den_t, den_h.at[tid])

    return k(src_e, dst_e, tsrc, tdst, z8f)


def _edge_agg(src_e, dst_e, table, p, rden, zF, Fh):
    N, F = table.shape
    E = src_e.shape[0]
    NW = _NC * _NS
    ept = E // NW
    nchunks = ept // _B
    rpt = (N // _NS) // 8 * 8
    tail = N - _NS * rpt
    nj = F // 16

    @functools.partial(
        pl.kernel,
        out_type=jax.ShapeDtypeStruct((_NC, N, F), jnp.float32),
        mesh=plsc.VectorSubcoreMesh(core_axis_name="c", subcore_axis_name="s"),
        compiler_params=pltpu.CompilerParams(
            needs_layout_passes=False, use_tc_tiling_on_sc=False),
        scratch_types=[
            pltpu.VMEM((4, _B), jnp.int32),
            pltpu.VMEM((4, _B), jnp.int32),
            pltpu.VMEM((2 * _B, F), jnp.float32),
            pltpu.VMEM((2 * _B, 8), jnp.float32),
            pltpu.VMEM((2 * _B, 8), jnp.float32),
            pltpu.VMEM((2 * _B, F), jnp.float32),
            pltpu.VMEM_SHARED((N, F), jnp.float32),
            pltpu.SemaphoreType.DMA,
            pltpu.SemaphoreType.DMA,
            pltpu.SemaphoreType.DMA,
            pltpu.SemaphoreType.DMA,
            pltpu.SemaphoreType.DMA,
            pltpu.SemaphoreType.DMA,
            pltpu.SemaphoreType.DMA,
            pltpu.SemaphoreType.DMA,
        ],
    )
    def k(srce_h, dste_h, tab_h, p_h, rden_h, zf_h, out_h,
          srcr, dstr, h_v, p_v, r_v, m_v, acc_sh,
          si0, si1, si2, si3, sg0, sg1, ss0, ss1):
        si = [si0, si1, si2, si3]
        sg = [sg0, sg1]
        ss = [ss0, ss1]
        cid = lax.axis_index("c")
        sid = lax.axis_index("s")
        tid = cid * _NS + sid
        r0 = sid * rpt
        pltpu.sync_copy(zf_h.at[pl.ds(r0, rpt)], acc_sh.at[pl.ds(r0, rpt)])

        @pl.when(sid == _NS - 1)
        def _():
            pltpu.sync_copy(zf_h.at[pl.ds(_NS * rpt, tail)],
                            acc_sh.at[pl.ds(_NS * rpt, tail)])

        plsc.subcore_barrier()

        iot = lax.iota(jnp.int32, 16)
        icol = lax.rem(iot, 8)
        irow = lax.div(iot, 8)
        jps = [lax.div(16 * j + iot, Fh) for j in range(nj)]
        cidxs = [16 * j + iot for j in range(nj)]

        def idx_issue(c, q):
            base = tid * ept + c * _B
            pltpu.async_copy(srce_h.at[pl.ds(base, _B)], srcr.at[q], si[q])
            pltpu.async_copy(dste_h.at[pl.ds(base, _B)], dstr.at[q], si[q])

        def idx_wait(c, q):
            base = tid * ept + c * _B
            pltpu.make_async_copy(
                srce_h.at[pl.ds(base, _B)], srcr.at[q], si[q]).wait()
            pltpu.make_async_copy(
                dste_h.at[pl.ds(base, _B)], dstr.at[q], si[q]).wait()

        def gat_issue2(c, b, q):
            base = tid * ept + c * _B
            pltpu.async_copy(tab_h.at[srcr.at[q]],
                             h_v.at[pl.ds(b * _B, _B)], sg[b])
            pltpu.async_copy(rden_h.at[dstr.at[q]],
                             r_v.at[pl.ds(b * _B, _B)], sg[b])
            pltpu.async_copy(p_h.at[pl.ds(base, _B)],
                             p_v.at[pl.ds(b * _B, _B)], sg[b])

        def gat_wait2(c, b, q):
            base = tid * ept + c * _B
            pltpu.make_async_copy(tab_h.at[srcr.at[q]],
                                  h_v.at[pl.ds(b * _B, _B)], sg[b]).wait()
            pltpu.make_async_copy(rden_h.at[dstr.at[q]],
                                  r_v.at[pl.ds(b * _B, _B)], sg[b]).wait()
            pltpu.make_async_copy(p_h.at[pl.ds(base, _B)],
                                  p_v.at[pl.ds(b * _B, _B)], sg[b]).wait()

        def compute(c, b):
            boff = b * _B

            def vb(j, c2):
                ia = boff + 2 * j + irow
                a = (plsc.load_gather(p_v, [ia, icol])
                     * plsc.load_gather(r_v, [ia, icol]))
                plsc.store_scatter(p_v, [ia, icol], a)
                return c2

            lax.fori_loop(0, _B // 2, vb, 0)

            def edge_body(kk, c2):
                row = boff + kk
                sk = jnp.full((16,), row, jnp.int32)
                for j in range(nj):
                    al = plsc.load_gather(p_v, [sk, jps[j]])
                    hv = h_v[row, pl.ds(16 * j, 16)]
                    m_v[row, pl.ds(16 * j, 16)] = hv * al
                return c2

            lax.fori_loop(0, _B, edge_body, 0)

        def st_issue2(c, b, q):
            pltpu.sync_copy(m_v.at[pl.ds(b * _B, _B)],
                            acc_sh.at[dstr.at[q]], add=True)

        def st_wait2(c, b, q):
            pass

        def step(c, r, n1, n2, prev):
            if n1:
                idx_wait(c + 1, (r + 1) % 4)
            if n2:
                idx_issue(c + 2, (r + 2) % 4)
            gat_wait2(c, r % 2, r)
            if n1:
                gat_issue2(c + 1, (r + 1) % 2, (r + 1) % 4)
            compute(c, r % 2)
            if prev:
                st_wait2(c - 1, (r - 1) % 2, (r - 1) % 4)
            st_issue2(c, r % 2, r)

        idx_issue(0, 0)
        idx_wait(0, 0)
        gat_issue2(0, 0, 0)
        idx_issue(1, 1)
        step(0, 0, True, True, False)

        def quad(t, carry):
            for cc in range(4):
                step(4 * t + 1 + cc, (1 + cc) % 4, True, True, True)
            return carry

        lax.fori_loop(0, (nchunks - 5) // 4, quad, 0)
        for c in range(nchunks - 4, nchunks):
            step(c, c % 4, c + 1 < nchunks, c + 2 < nchunks, True)
        st_wait2(nchunks - 1, (nchunks - 1) % 2, (nchunks - 1) % 4)

        plsc.subcore_barrier()
        pltpu.sync_copy(acc_sh.at[pl.ds(r0, rpt)],
                        out_h.at[cid, pl.ds(r0, rpt)])

        @pl.when(sid == _NS - 1)
        def _():
            pltpu.sync_copy(acc_sh.at[pl.ds(_NS * rpt, tail)],
                            out_h.at[cid, pl.ds(_NS * rpt, tail)])

    return k(src_e, dst_e, table, p, rden, zF)


_BN = 1000


def _proj1(x, W1, As, Ad):
    N, Fin = x.shape
    Fo = W1.shape[1]
    H = As.shape[1]

    def body(x_r, w_r, as_r, ad_r, h_r, ts_r, td_r):
        h = jnp.dot(x_r[...], w_r[...], preferred_element_type=jnp.float32)
        h_r[...] = h
        ts_r[...] = jnp.dot(h, as_r[...], preferred_element_type=jnp.float32)
        td_r[...] = jnp.dot(h, ad_r[...], preferred_element_type=jnp.float32)

    return pl.pallas_call(
        body,
        grid=(N // _BN,),
        in_specs=[
            pl.BlockSpec((_BN, Fin), lambda i: (i, 0)),
            pl.BlockSpec((Fin, Fo), lambda i: (0, 0)),
            pl.BlockSpec((Fo, H), lambda i: (0, 0)),
            pl.BlockSpec((Fo, H), lambda i: (0, 0)),
        ],
        out_specs=[
            pl.BlockSpec((_BN, Fo), lambda i: (i, 0)),
            pl.BlockSpec((_BN, H), lambda i: (i, 0)),
            pl.BlockSpec((_BN, H), lambda i: (i, 0)),
        ],
        out_shape=[
            jax.ShapeDtypeStruct((N, Fo), jnp.float32),
            jax.ShapeDtypeStruct((N, H), jnp.float32),
            jax.ShapeDtypeStruct((N, H), jnp.float32),
        ],
    )(x, W1, As, Ad)


def _rden(den):
    T, N, H = den.shape

    def body(d_r, r_r):
        r_r[...] = 1.0 / (jnp.sum(d_r[...], axis=0) + 1e-16)

    return pl.pallas_call(
        body,
        grid=(N // _BN,),
        in_specs=[pl.BlockSpec((T, _BN, H), lambda i: (0, i, 0))],
        out_specs=pl.BlockSpec((_BN, H), lambda i: (i, 0)),
        out_shape=jax.ShapeDtypeStruct((N, H), jnp.float32),
    )(den)


def _layer2_prep(o, b1, W2, As2, Ad2):
    _, N, Fo = o.shape
    C = W2.shape[1]

    def body(o_r, b_r, w_r, as_r, ad_r, h2_r, ts_r, td_r):
        t = o_r[0] + o_r[1] + b_r[...]
        t = jnp.where(t > 0, t, jnp.exp(t) - 1.0)
        h2 = jnp.dot(t, w_r[...], preferred_element_type=jnp.float32)
        h2_r[...] = h2
        ts_r[...] = jnp.dot(h2, as_r[...], preferred_element_type=jnp.float32)
        td_r[...] = jnp.dot(h2, ad_r[...], preferred_element_type=jnp.float32)

    return pl.pallas_call(
        body,
        grid=(N // _BN,),
        in_specs=[
            pl.BlockSpec((2, _BN, Fo), lambda i: (0, i, 0)),
            pl.BlockSpec((1, Fo), lambda i: (0, 0)),
            pl.BlockSpec((Fo, C), lambda i: (0, 0)),
            pl.BlockSpec((C, 8), lambda i: (0, 0)),
            pl.BlockSpec((C, 8), lambda i: (0, 0)),
        ],
        out_specs=[
            pl.BlockSpec((_BN, C), lambda i: (i, 0)),
            pl.BlockSpec((_BN, 8), lambda i: (i, 0)),
            pl.BlockSpec((_BN, 8), lambda i: (i, 0)),
        ],
        out_shape=[
            jax.ShapeDtypeStruct((N, C), jnp.float32),
            jax.ShapeDtypeStruct((N, 8), jnp.float32),
            jax.ShapeDtypeStruct((N, 8), jnp.float32),
        ],
    )(o, b1, W2, As2, Ad2)


def _final(o, b2):
    _, N, C = o.shape

    def body(o_r, b_r, out_r):
        t = o_r[0] + o_r[1] + b_r[...]
        m = jnp.max(t, axis=1, keepdims=True)
        te = t - m
        lse = jnp.log(jnp.sum(jnp.exp(te), axis=1, keepdims=True))
        out_r[...] = te - lse

    return pl.pallas_call(
        body,
        grid=(N // _BN,),
        in_specs=[
            pl.BlockSpec((2, _BN, C), lambda i: (0, i, 0)),
            pl.BlockSpec((1, C), lambda i: (0, 0)),
        ],
        out_specs=pl.BlockSpec((_BN, C), lambda i: (i, 0)),
        out_shape=jax.ShapeDtypeStruct((N, C), jnp.float32),
    )(o, b2)


def kernel(x, edge_index, W1, a_src1, a_dst1, b1, W2, a_src2, a_dst2, b2):
    N, _ = x.shape
    H1, F1 = a_src1.shape
    C = W2.shape[1]

    eye = jnp.eye(H1, dtype=jnp.float32)
    As1 = (a_src1[:, :, None] * eye[:, None, :]).reshape(H1 * F1, H1)
    Ad1 = (a_dst1[:, :, None] * eye[:, None, :]).reshape(H1 * F1, H1)
    As2 = jnp.zeros((C, 8), jnp.float32).at[:, 0].set(a_src2[0])
    Ad2 = jnp.zeros((C, 8), jnp.float32).at[:, 0].set(a_dst2[0])

    z8f = jnp.zeros((N * 8,), jnp.float32)
    z64 = jnp.zeros((N, H1 * F1), jnp.float32)
    z16 = jnp.zeros((N, C), jnp.float32)

    src_e = edge_index[0]
    dst_e = edge_index[1]
    NW = _NC * _NS

    h1, ts1, td1 = _proj1(x, W1, As1, Ad1)
    p1, den1 = _edge_den(src_e, dst_e, ts1, td1, z8f)
    rden1 = _rden(den1.reshape(NW, N, 8))
    o1 = _edge_agg(src_e, dst_e, h1, p1, rden1, z64, Fh=F1)
    h2, ts2, td2 = _layer2_prep(o1, b1.reshape(1, -1), W2, As2, Ad2)
    p2, den2 = _edge_den(src_e, dst_e, ts2, td2, z8f)
    rden2 = _rden(den2.reshape(NW, N, 8))
    o2 = _edge_agg(src_e, dst_e, h2, p2, rden2, z16, Fh=C)
    return _final(o2, b2.reshape(1, -1))


# speedup vs baseline: 1.3074x; 1.3074x over previous
import functools

import jax
import jax.numpy as jnp
from jax import lax
from jax.experimental import pallas as pl
from jax.experimental.pallas import tpu as pltpu
from jax.experimental.pallas import tpu_sc as plsc

try:
    _info = plsc.get_sparse_core_info()
    _NC, _NS = int(_info.num_cores), int(_info.num_subcores)
except Exception:
    _NC, _NS = 2, 16

_B = 80


def _edge_den(src_e, dst_e, tsrc, tdst, z8):
    N = tsrc.shape[0]
    E = src_e.shape[0]
    NW = _NC * _NS
    ept = E // NW
    nchunks = ept // _B
    rpt = (N // _NS) // 8 * 8
    tail = N - _NS * rpt

    @functools.partial(
        pl.kernel,
        out_type=(
            jax.ShapeDtypeStruct((E, 8), jnp.float32),
            jax.ShapeDtypeStruct((_NC, N, 8), jnp.float32),
        ),
        mesh=plsc.VectorSubcoreMesh(core_axis_name="c", subcore_axis_name="s"),
        compiler_params=pltpu.CompilerParams(
            needs_layout_passes=False, use_tc_tiling_on_sc=False),
        scratch_types=[
            pltpu.VMEM((4, _B), jnp.int32),
            pltpu.VMEM((4, _B), jnp.int32),
            pltpu.VMEM((2 * _B, 8), jnp.float32),
            pltpu.VMEM((2 * _B, 8), jnp.float32),
            pltpu.VMEM((2 * _B, 8), jnp.float32),
            pltpu.VMEM_SHARED((N, 8), jnp.float32),
            pltpu.SemaphoreType.DMA,
            pltpu.SemaphoreType.DMA,
            pltpu.SemaphoreType.DMA,
            pltpu.SemaphoreType.DMA,
            pltpu.SemaphoreType.DMA,
            pltpu.SemaphoreType.DMA,
            pltpu.SemaphoreType.DMA,
            pltpu.SemaphoreType.DMA,
        ],
    )
    def k(srce_h, dste_h, tsrc_h, tdst_h, z8_h, p_h, den_h,
          srcr, dstr, as_v, ad_v, p_v, den_sh,
          si0, si1, si2, si3, sg0, sg1, ss0, ss1):
        si = [si0, si1, si2, si3]
        sg = [sg0, sg1]
        ss = [ss0, ss1]
        cid = lax.axis_index("c")
        sid = lax.axis_index("s")
        tid = cid * _NS + sid
        r0 = sid * rpt
        pltpu.sync_copy(z8_h.at[pl.ds(r0, rpt)], den_sh.at[pl.ds(r0, rpt)])

        @pl.when(sid == _NS - 1)
        def _():
            pltpu.sync_copy(z8_h.at[pl.ds(_NS * rpt, tail)],
                            den_sh.at[pl.ds(_NS * rpt, tail)])

        plsc.subcore_barrier()

        iot = lax.iota(jnp.int32, 16)
        icol = lax.rem(iot, 8)
        irow = lax.div(iot, 8)

        def idx_issue(c, q):
            base = tid * ept + c * _B
            pltpu.async_copy(srce_h.at[pl.ds(base, _B)], srcr.at[q], si[q])
            pltpu.async_copy(dste_h.at[pl.ds(base, _B)], dstr.at[q], si[q])

        def idx_wait(c, q):
            base = tid * ept + c * _B
            pltpu.make_async_copy(
                srce_h.at[pl.ds(base, _B)], srcr.at[q], si[q]).wait()
            pltpu.make_async_copy(
                dste_h.at[pl.ds(base, _B)], dstr.at[q], si[q]).wait()

        def gat_issue2(c, b, q):
            pltpu.async_copy(tsrc_h.at[srcr.at[q]],
                             as_v.at[pl.ds(b * _B, _B)], sg[b])
            pltpu.async_copy(tdst_h.at[dstr.at[q]],
                             ad_v.at[pl.ds(b * _B, _B)], sg[b])

        def gat_wait2(c, b, q):
            pltpu.make_async_copy(tsrc_h.at[srcr.at[q]],
                                  as_v.at[pl.ds(b * _B, _B)], sg[b]).wait()
            pltpu.make_async_copy(tdst_h.at[dstr.at[q]],
                                  ad_v.at[pl.ds(b * _B, _B)], sg[b]).wait()

        def compute(c, b, q):
            boff = b * _B

            def vb(j, c2):
                ia = boff + 2 * j + irow
                s = (plsc.load_gather(as_v, [ia, icol])
                     + plsc.load_gather(ad_v, [ia, icol]))
                p = jnp.exp(jnp.maximum(s, 0.2 * s))
                plsc.store_scatter(p_v, [ia, icol], p)
                return c2

            lax.fori_loop(0, _B // 2, vb, 0)

        def st_issue2(c, b, q):
            base = tid * ept + c * _B
            pltpu.async_copy(p_v.at[pl.ds(b * _B, _B)],
                             p_h.at[pl.ds(base, _B)], ss[b])
            pltpu.sync_copy(p_v.at[pl.ds(b * _B, _B)],
                            den_sh.at[dstr.at[q]], add=True)

        def st_wait2(c, b, q):
            base = tid * ept + c * _B
            pltpu.make_async_copy(p_v.at[pl.ds(b * _B, _B)],
                                  p_h.at[pl.ds(base, _B)], ss[b]).wait()

        def step(c, r, n1, n2, prev):
            if n1:
                idx_wait(c + 1, (r + 1) % 4)
            if n2:
                idx_issue(c + 2, (r + 2) % 4)
            gat_wait2(c, r % 2, r)
            if n1:
                gat_issue2(c + 1, (r + 1) % 2, (r + 1) % 4)
            compute(c, r % 2, r)
            if prev:
                st_wait2(c - 1, (r - 1) % 2, (r - 1) % 4)
            st_issue2(c, r % 2, r)

        idx_issue(0, 0)
        idx_wait(0, 0)
        gat_issue2(0, 0, 0)
        idx_issue(1, 1)
        step(0, 0, True, True, False)

        def quad(t, carry):
            for cc in range(4):
                step(4 * t + 1 + cc, (1 + cc) % 4, True, True, True)
            return carry

        lax.fori_loop(0, (nchunks - 5) // 4, quad, 0)
        for c in range(nchunks - 4, nchunks):
            step(c, c % 4, c + 1 < nchunks, c + 2 < nchunks, True)
        st_wait2(nchunks - 1, (nchunks - 1) % 2, (nchunks - 1) % 4)

        plsc.subcore_barrier()
        pltpu.sync_copy(den_sh.at[pl.ds(r0, rpt)],
                        den_h.at[cid, pl.ds(r0, rpt)])

        @pl.when(sid == _NS - 1)
        def _():
            pltpu.sync_copy(den_sh.at[pl.ds(_NS * rpt, tail)],
                            den_h.at[cid, pl.ds(_NS * rpt, tail)])

    return k(src_e, dst_e, tsrc, tdst, z8)


def _edge_agg(src_e, dst_e, table, p, den0, den1, zF, Fh):
    N, F = table.shape
    E = src_e.shape[0]
    NW = _NC * _NS
    ept = E // NW
    nchunks = ept // _B
    rpt = (N // _NS) // 8 * 8
    tail = N - _NS * rpt
    nj = F // 16

    @functools.partial(
        pl.kernel,
        out_type=jax.ShapeDtypeStruct((_NC, N, F), jnp.float32),
        mesh=plsc.VectorSubcoreMesh(core_axis_name="c", subcore_axis_name="s"),
        compiler_params=pltpu.CompilerParams(
            needs_layout_passes=False, use_tc_tiling_on_sc=False),
        scratch_types=[
            pltpu.VMEM((4, _B), jnp.int32),
            pltpu.VMEM((4, _B), jnp.int32),
            pltpu.VMEM((2 * _B, F), jnp.float32),
            pltpu.VMEM((2 * _B, 8), jnp.float32),
            pltpu.VMEM((2 * _B, 8), jnp.float32),
            pltpu.VMEM((2 * _B, 8), jnp.float32),
            pltpu.VMEM((2 * _B, F), jnp.float32),
            pltpu.VMEM_SHARED((N, F), jnp.float32),
            pltpu.SemaphoreType.DMA,
            pltpu.SemaphoreType.DMA,
            pltpu.SemaphoreType.DMA,
            pltpu.SemaphoreType.DMA,
            pltpu.SemaphoreType.DMA,
            pltpu.SemaphoreType.DMA,
            pltpu.SemaphoreType.DMA,
            pltpu.SemaphoreType.DMA,
        ],
    )
    def k(srce_h, dste_h, tab_h, p_h, den0_h, den1_h, zf_h, out_h,
          srcr, dstr, h_v, p_v, r_v, r1_v, m_v, acc_sh,
          si0, si1, si2, si3, sg0, sg1, ss0, ss1):
        si = [si0, si1, si2, si3]
        sg = [sg0, sg1]
        ss = [ss0, ss1]
        cid = lax.axis_index("c")
        sid = lax.axis_index("s")
        tid = cid * _NS + sid
        r0 = sid * rpt
        pltpu.sync_copy(zf_h.at[pl.ds(r0, rpt)], acc_sh.at[pl.ds(r0, rpt)])

        @pl.when(sid == _NS - 1)
        def _():
            pltpu.sync_copy(zf_h.at[pl.ds(_NS * rpt, tail)],
                            acc_sh.at[pl.ds(_NS * rpt, tail)])

        plsc.subcore_barrier()

        iot = lax.iota(jnp.int32, 16)
        icol = lax.rem(iot, 8)
        irow = lax.div(iot, 8)
        jps = [lax.div(16 * j + iot, Fh) for j in range(nj)]
        cidxs = [16 * j + iot for j in range(nj)]

        def idx_issue(c, q):
            base = tid * ept + c * _B
            pltpu.async_copy(srce_h.at[pl.ds(base, _B)], srcr.at[q], si[q])
            pltpu.async_copy(dste_h.at[pl.ds(base, _B)], dstr.at[q], si[q])

        def idx_wait(c, q):
            base = tid * ept + c * _B
            pltpu.make_async_copy(
                srce_h.at[pl.ds(base, _B)], srcr.at[q], si[q]).wait()
            pltpu.make_async_copy(
                dste_h.at[pl.ds(base, _B)], dstr.at[q], si[q]).wait()

        def gat_issue2(c, b, q):
            base = tid * ept + c * _B
            pltpu.async_copy(tab_h.at[srcr.at[q]],
                             h_v.at[pl.ds(b * _B, _B)], sg[b])
            pltpu.async_copy(den0_h.at[dstr.at[q]],
                             r_v.at[pl.ds(b * _B, _B)], sg[b])
            pltpu.async_copy(den1_h.at[dstr.at[q]],
                             r1_v.at[pl.ds(b * _B, _B)], sg[b])
            pltpu.async_copy(p_h.at[pl.ds(base, _B)],
                             p_v.at[pl.ds(b * _B, _B)], sg[b])

        def gat_wait2(c, b, q):
            base = tid * ept + c * _B
            pltpu.make_async_copy(tab_h.at[srcr.at[q]],
                                  h_v.at[pl.ds(b * _B, _B)], sg[b]).wait()
            pltpu.make_async_copy(den0_h.at[dstr.at[q]],
                                  r_v.at[pl.ds(b * _B, _B)], sg[b]).wait()
            pltpu.make_async_copy(den1_h.at[dstr.at[q]],
                                  r1_v.at[pl.ds(b * _B, _B)], sg[b]).wait()
            pltpu.make_async_copy(p_h.at[pl.ds(base, _B)],
                                  p_v.at[pl.ds(b * _B, _B)], sg[b]).wait()

        def compute(c, b):
            boff = b * _B

            def vb(j, c2):
                ia = boff + 2 * j + irow
                den = (plsc.load_gather(r_v, [ia, icol])
                       + plsc.load_gather(r1_v, [ia, icol]) + 1e-16)
                a = plsc.load_gather(p_v, [ia, icol]) / den
                plsc.store_scatter(p_v, [ia, icol], a)
                return c2

            lax.fori_loop(0, _B // 2, vb, 0)

            def edge_body(kk, c2):
                row = boff + kk
                sk = jnp.full((16,), row, jnp.int32)
                for j in range(nj):
                    al = plsc.load_gather(p_v, [sk, jps[j]])
                    hv = h_v[row, pl.ds(16 * j, 16)]
                    m_v[row, pl.ds(16 * j, 16)] = hv * al
                return c2

            lax.fori_loop(0, _B, edge_body, 0)

        def st_issue2(c, b, q):
            pltpu.sync_copy(m_v.at[pl.ds(b * _B, _B)],
                            acc_sh.at[dstr.at[q]], add=True)

        def st_wait2(c, b, q):
            pass

        def step(c, r, n1, n2, prev):
            if n1:
                idx_wait(c + 1, (r + 1) % 4)
            if n2:
                idx_issue(c + 2, (r + 2) % 4)
            gat_wait2(c, r % 2, r)
            if n1:
                gat_issue2(c + 1, (r + 1) % 2, (r + 1) % 4)
            compute(c, r % 2)
            if prev:
                st_wait2(c - 1, (r - 1) % 2, (r - 1) % 4)
            st_issue2(c, r % 2, r)

        idx_issue(0, 0)
        idx_wait(0, 0)
        gat_issue2(0, 0, 0)
        idx_issue(1, 1)
        step(0, 0, True, True, False)

        def quad(t, carry):
            for cc in range(4):
                step(4 * t + 1 + cc, (1 + cc) % 4, True, True, True)
            return carry

        lax.fori_loop(0, (nchunks - 5) // 4, quad, 0)
        for c in range(nchunks - 4, nchunks):
            step(c, c % 4, c + 1 < nchunks, c + 2 < nchunks, True)
        st_wait2(nchunks - 1, (nchunks - 1) % 2, (nchunks - 1) % 4)

        plsc.subcore_barrier()
        pltpu.sync_copy(acc_sh.at[pl.ds(r0, rpt)],
                        out_h.at[cid, pl.ds(r0, rpt)])

        @pl.when(sid == _NS - 1)
        def _():
            pltpu.sync_copy(acc_sh.at[pl.ds(_NS * rpt, tail)],
                            out_h.at[cid, pl.ds(_NS * rpt, tail)])

    return k(src_e, dst_e, table, p, den0, den1, zF)


_BN = 1000


def _proj1(x, W1, As, Ad):
    N, Fin = x.shape
    Fo = W1.shape[1]
    H = As.shape[1]

    def body(x_r, w_r, as_r, ad_r, h_r, ts_r, td_r):
        h = jnp.dot(x_r[...], w_r[...], preferred_element_type=jnp.float32)
        h_r[...] = h
        ts_r[...] = jnp.dot(h, as_r[...], preferred_element_type=jnp.float32)
        td_r[...] = jnp.dot(h, ad_r[...], preferred_element_type=jnp.float32)

    return pl.pallas_call(
        body,
        grid=(N // _BN,),
        in_specs=[
            pl.BlockSpec((_BN, Fin), lambda i: (i, 0)),
            pl.BlockSpec((Fin, Fo), lambda i: (0, 0)),
            pl.BlockSpec((Fo, H), lambda i: (0, 0)),
            pl.BlockSpec((Fo, H), lambda i: (0, 0)),
        ],
        out_specs=[
            pl.BlockSpec((_BN, Fo), lambda i: (i, 0)),
            pl.BlockSpec((_BN, H), lambda i: (i, 0)),
            pl.BlockSpec((_BN, H), lambda i: (i, 0)),
        ],
        out_shape=[
            jax.ShapeDtypeStruct((N, Fo), jnp.float32),
            jax.ShapeDtypeStruct((N, H), jnp.float32),
            jax.ShapeDtypeStruct((N, H), jnp.float32),
        ],
    )(x, W1, As, Ad)


def _rden(den):
    T, N, H = den.shape

    def body(d_r, r_r):
        r_r[...] = 1.0 / (jnp.sum(d_r[...], axis=0) + 1e-16)

    return pl.pallas_call(
        body,
        grid=(N // _BN,),
        in_specs=[pl.BlockSpec((T, _BN, H), lambda i: (0, i, 0))],
        out_specs=pl.BlockSpec((_BN, H), lambda i: (i, 0)),
        out_shape=jax.ShapeDtypeStruct((N, H), jnp.float32),
    )(den)


def _layer2_prep(o, b1, W2, As2, Ad2):
    _, N, Fo = o.shape
    C = W2.shape[1]

    def body(o_r, b_r, w_r, as_r, ad_r, h2_r, ts_r, td_r):
        t = o_r[0] + o_r[1] + b_r[...]
        t = jnp.where(t > 0, t, jnp.exp(t) - 1.0)
        h2 = jnp.dot(t, w_r[...], preferred_element_type=jnp.float32)
        h2_r[...] = h2
        ts_r[...] = jnp.dot(h2, as_r[...], preferred_element_type=jnp.float32)
        td_r[...] = jnp.dot(h2, ad_r[...], preferred_element_type=jnp.float32)

    return pl.pallas_call(
        body,
        grid=(N // _BN,),
        in_specs=[
            pl.BlockSpec((2, _BN, Fo), lambda i: (0, i, 0)),
            pl.BlockSpec((1, Fo), lambda i: (0, 0)),
            pl.BlockSpec((Fo, C), lambda i: (0, 0)),
            pl.BlockSpec((C, 8), lambda i: (0, 0)),
            pl.BlockSpec((C, 8), lambda i: (0, 0)),
        ],
        out_specs=[
            pl.BlockSpec((_BN, C), lambda i: (i, 0)),
            pl.BlockSpec((_BN, 8), lambda i: (i, 0)),
            pl.BlockSpec((_BN, 8), lambda i: (i, 0)),
        ],
        out_shape=[
            jax.ShapeDtypeStruct((N, C), jnp.float32),
            jax.ShapeDtypeStruct((N, 8), jnp.float32),
            jax.ShapeDtypeStruct((N, 8), jnp.float32),
        ],
    )(o, b1, W2, As2, Ad2)


def _final(o, b2):
    _, N, C = o.shape

    def body(o_r, b_r, out_r):
        t = o_r[0] + o_r[1] + b_r[...]
        m = jnp.max(t, axis=1, keepdims=True)
        te = t - m
        lse = jnp.log(jnp.sum(jnp.exp(te), axis=1, keepdims=True))
        out_r[...] = te - lse

    return pl.pallas_call(
        body,
        grid=(N // _BN,),
        in_specs=[
            pl.BlockSpec((2, _BN, C), lambda i: (0, i, 0)),
            pl.BlockSpec((1, C), lambda i: (0, 0)),
        ],
        out_specs=pl.BlockSpec((_BN, C), lambda i: (i, 0)),
        out_shape=jax.ShapeDtypeStruct((N, C), jnp.float32),
    )(o, b2)


def kernel(x, edge_index, W1, a_src1, a_dst1, b1, W2, a_src2, a_dst2, b2):
    N, _ = x.shape
    H1, F1 = a_src1.shape
    C = W2.shape[1]

    eye = jnp.eye(H1, dtype=jnp.float32)
    As1 = (a_src1[:, :, None] * eye[:, None, :]).reshape(H1 * F1, H1)
    Ad1 = (a_dst1[:, :, None] * eye[:, None, :]).reshape(H1 * F1, H1)
    As2 = jnp.zeros((C, 8), jnp.float32).at[:, 0].set(a_src2[0])
    Ad2 = jnp.zeros((C, 8), jnp.float32).at[:, 0].set(a_dst2[0])

    z8 = jnp.zeros((N, 8), jnp.float32)
    z64 = jnp.zeros((N, H1 * F1), jnp.float32)
    z16 = jnp.zeros((N, C), jnp.float32)

    src_e = edge_index[0]
    dst_e = edge_index[1]

    h1, ts1, td1 = _proj1(x, W1, As1, Ad1)
    p1, den1 = _edge_den(src_e, dst_e, ts1, td1, z8)
    o1 = _edge_agg(src_e, dst_e, h1, p1, den1[0], den1[1], z64, Fh=F1)
    h2, ts2, td2 = _layer2_prep(o1, b1.reshape(1, -1), W2, As2, Ad2)
    p2, den2 = _edge_den(src_e, dst_e, ts2, td2, z8)
    o2 = _edge_agg(src_e, dst_e, h2, p2, den2[0], den2[1], z16, Fh=C)
    return _final(o2, b2.reshape(1, -1))


# speedup vs baseline: 1.4380x; 1.0999x over previous
import functools

import jax
import jax.numpy as jnp
from jax import lax
from jax.experimental import pallas as pl
from jax.experimental.pallas import tpu as pltpu
from jax.experimental.pallas import tpu_sc as plsc

try:
    _info = plsc.get_sparse_core_info()
    _NC, _NS = int(_info.num_cores), int(_info.num_subcores)
except Exception:
    _NC, _NS = 2, 16

_B = 80


def _edge_den(src_e, dst_e, tsrc, tdst, z8):
    N = tsrc.shape[0]
    E = src_e.shape[0]
    NW = _NC * _NS
    ept = E // NW
    nchunks = ept // _B
    rpt = (N // _NS) // 8 * 8
    tail = N - _NS * rpt

    @functools.partial(
        pl.kernel,
        out_type=(
            jax.ShapeDtypeStruct((E, 8), jnp.float32),
            jax.ShapeDtypeStruct((_NC, N, 8), jnp.float32),
        ),
        mesh=plsc.VectorSubcoreMesh(core_axis_name="c", subcore_axis_name="s"),
        compiler_params=pltpu.CompilerParams(
            needs_layout_passes=False, use_tc_tiling_on_sc=False),
        scratch_types=[
            pltpu.VMEM((4, _B), jnp.int32),
            pltpu.VMEM((4, _B), jnp.int32),
            pltpu.VMEM((2 * _B, 8), jnp.float32),
            pltpu.VMEM((2 * _B, 8), jnp.float32),
            pltpu.VMEM((2 * _B, 8), jnp.float32),
            pltpu.VMEM_SHARED((N, 8), jnp.float32),
            pltpu.SemaphoreType.DMA,
            pltpu.SemaphoreType.DMA,
            pltpu.SemaphoreType.DMA,
            pltpu.SemaphoreType.DMA,
            pltpu.SemaphoreType.DMA,
            pltpu.SemaphoreType.DMA,
            pltpu.SemaphoreType.DMA,
            pltpu.SemaphoreType.DMA,
        ],
    )
    def k(srce_h, dste_h, tsrc_h, tdst_h, z8_h, p_h, den_h,
          srcr, dstr, as_v, ad_v, p_v, den_sh,
          si0, si1, si2, si3, sg0, sg1, ss0, ss1):
        si = [si0, si1, si2, si3]
        sg = [sg0, sg1]
        ss = [ss0, ss1]
        cid = lax.axis_index("c")
        sid = lax.axis_index("s")
        tid = cid * _NS + sid
        r0 = sid * rpt
        pltpu.sync_copy(z8_h.at[pl.ds(r0, rpt)], den_sh.at[pl.ds(r0, rpt)])

        @pl.when(sid == _NS - 1)
        def _():
            pltpu.sync_copy(z8_h.at[pl.ds(_NS * rpt, tail)],
                            den_sh.at[pl.ds(_NS * rpt, tail)])

        plsc.subcore_barrier()

        iot = lax.iota(jnp.int32, 16)
        icol = lax.rem(iot, 8)
        irow = lax.div(iot, 8)

        def idx_issue(c, q):
            base = tid * ept + c * _B
            pltpu.async_copy(srce_h.at[pl.ds(base, _B)], srcr.at[q], si[q])
            pltpu.async_copy(dste_h.at[pl.ds(base, _B)], dstr.at[q], si[q])

        def idx_wait(c, q):
            base = tid * ept + c * _B
            pltpu.make_async_copy(
                srce_h.at[pl.ds(base, _B)], srcr.at[q], si[q]).wait()
            pltpu.make_async_copy(
                dste_h.at[pl.ds(base, _B)], dstr.at[q], si[q]).wait()

        def gat_issue2(c, b, q):
            pltpu.async_copy(tsrc_h.at[srcr.at[q]],
                             as_v.at[pl.ds(b * _B, _B)], sg[b])
            pltpu.async_copy(tdst_h.at[dstr.at[q]],
                             ad_v.at[pl.ds(b * _B, _B)], sg[b])

        def gat_wait2(c, b, q):
            pltpu.make_async_copy(tsrc_h.at[srcr.at[q]],
                                  as_v.at[pl.ds(b * _B, _B)], sg[b]).wait()
            pltpu.make_async_copy(tdst_h.at[dstr.at[q]],
                                  ad_v.at[pl.ds(b * _B, _B)], sg[b]).wait()

        def compute(c, b, q):
            boff = b * _B

            def vb(j, c2):
                ia = boff + 2 * j + irow
                s = (plsc.load_gather(as_v, [ia, icol])
                     + plsc.load_gather(ad_v, [ia, icol]))
                p = jnp.exp(jnp.maximum(s, 0.2 * s))
                plsc.store_scatter(p_v, [ia, icol], p)
                return c2

            lax.fori_loop(0, _B // 2, vb, 0)

        def st_issue2(c, b, q):
            base = tid * ept + c * _B
            pltpu.async_copy(p_v.at[pl.ds(b * _B, _B)],
                             p_h.at[pl.ds(base, _B)], ss[b])
            pltpu.sync_copy(p_v.at[pl.ds(b * _B, _B)],
                            den_sh.at[dstr.at[q]], add=True)

        def st_wait2(c, b, q):
            base = tid * ept + c * _B
            pltpu.make_async_copy(p_v.at[pl.ds(b * _B, _B)],
                                  p_h.at[pl.ds(base, _B)], ss[b]).wait()

        def step(c, r, n1, n2, prev):
            if n1:
                idx_wait(c + 1, (r + 1) % 4)
            if n2:
                idx_issue(c + 2, (r + 2) % 4)
            gat_wait2(c, r % 2, r)
            if n1:
                gat_issue2(c + 1, (r + 1) % 2, (r + 1) % 4)
            compute(c, r % 2, r)
            if prev:
                st_wait2(c - 1, (r - 1) % 2, (r - 1) % 4)
            st_issue2(c, r % 2, r)

        idx_issue(0, 0)
        idx_wait(0, 0)
        gat_issue2(0, 0, 0)
        idx_issue(1, 1)
        step(0, 0, True, True, False)

        def quad(t, carry):
            for cc in range(4):
                step(4 * t + 1 + cc, (1 + cc) % 4, True, True, True)
            return carry

        lax.fori_loop(0, (nchunks - 5) // 4, quad, 0)
        for c in range(nchunks - 4, nchunks):
            step(c, c % 4, c + 1 < nchunks, c + 2 < nchunks, True)
        st_wait2(nchunks - 1, (nchunks - 1) % 2, (nchunks - 1) % 4)

        plsc.subcore_barrier()
        pltpu.sync_copy(den_sh.at[pl.ds(r0, rpt)],
                        den_h.at[cid, pl.ds(r0, rpt)])

        @pl.when(sid == _NS - 1)
        def _():
            pltpu.sync_copy(den_sh.at[pl.ds(_NS * rpt, tail)],
                            den_h.at[cid, pl.ds(_NS * rpt, tail)])

    return k(src_e, dst_e, tsrc, tdst, z8)


def _edge_agg(src_e, dst_e, table, p, rden, zF, Fh):
    N, F = table.shape
    E = src_e.shape[0]
    NW = _NC * _NS
    ept = E // NW
    nchunks = ept // _B
    rpt = (N // _NS) // 8 * 8
    tail = N - _NS * rpt
    nj = F // 16

    @functools.partial(
        pl.kernel,
        out_type=jax.ShapeDtypeStruct((_NC, N, F), jnp.float32),
        mesh=plsc.VectorSubcoreMesh(core_axis_name="c", subcore_axis_name="s"),
        compiler_params=pltpu.CompilerParams(
            needs_layout_passes=False, use_tc_tiling_on_sc=False),
        scratch_types=[
            pltpu.VMEM((4, _B), jnp.int32),
            pltpu.VMEM((4, _B), jnp.int32),
            pltpu.VMEM((2 * _B, F), jnp.float32),
            pltpu.VMEM((2 * _B, 8), jnp.float32),
            pltpu.VMEM((2 * _B, 8), jnp.float32),
            pltpu.VMEM((2 * _B, F), jnp.float32),
            pltpu.VMEM_SHARED((N, F), jnp.float32),
            pltpu.SemaphoreType.DMA,
            pltpu.SemaphoreType.DMA,
            pltpu.SemaphoreType.DMA,
            pltpu.SemaphoreType.DMA,
            pltpu.SemaphoreType.DMA,
            pltpu.SemaphoreType.DMA,
            pltpu.SemaphoreType.DMA,
            pltpu.SemaphoreType.DMA,
        ],
    )
    def k(srce_h, dste_h, tab_h, p_h, rden_h, zf_h, out_h,
          srcr, dstr, h_v, p_v, r_v, m_v, acc_sh,
          si0, si1, si2, si3, sg0, sg1, ss0, ss1):
        si = [si0, si1, si2, si3]
        sg = [sg0, sg1]
        ss = [ss0, ss1]
        cid = lax.axis_index("c")
        sid = lax.axis_index("s")
        tid = cid * _NS + sid
        r0 = sid * rpt
        pltpu.sync_copy(zf_h.at[pl.ds(r0, rpt)], acc_sh.at[pl.ds(r0, rpt)])

        @pl.when(sid == _NS - 1)
        def _():
            pltpu.sync_copy(zf_h.at[pl.ds(_NS * rpt, tail)],
                            acc_sh.at[pl.ds(_NS * rpt, tail)])

        plsc.subcore_barrier()

        iot = lax.iota(jnp.int32, 16)
        icol = lax.rem(iot, 8)
        irow = lax.div(iot, 8)
        jps = [lax.div(16 * j + iot, Fh) for j in range(nj)]
        cidxs = [16 * j + iot for j in range(nj)]

        def idx_issue(c, q):
            base = tid * ept + c * _B
            pltpu.async_copy(srce_h.at[pl.ds(base, _B)], srcr.at[q], si[q])
            pltpu.async_copy(dste_h.at[pl.ds(base, _B)], dstr.at[q], si[q])

        def idx_wait(c, q):
            base = tid * ept + c * _B
            pltpu.make_async_copy(
                srce_h.at[pl.ds(base, _B)], srcr.at[q], si[q]).wait()
            pltpu.make_async_copy(
                dste_h.at[pl.ds(base, _B)], dstr.at[q], si[q]).wait()

        def gat_issue2(c, b, q):
            base = tid * ept + c * _B
            pltpu.async_copy(tab_h.at[srcr.at[q]],
                             h_v.at[pl.ds(b * _B, _B)], sg[b])
            pltpu.async_copy(rden_h.at[dstr.at[q]],
                             r_v.at[pl.ds(b * _B, _B)], sg[b])
            pltpu.async_copy(p_h.at[pl.ds(base, _B)],
                             p_v.at[pl.ds(b * _B, _B)], sg[b])

        def gat_wait2(c, b, q):
            base = tid * ept + c * _B
            pltpu.make_async_copy(tab_h.at[srcr.at[q]],
                                  h_v.at[pl.ds(b * _B, _B)], sg[b]).wait()
            pltpu.make_async_copy(rden_h.at[dstr.at[q]],
                                  r_v.at[pl.ds(b * _B, _B)], sg[b]).wait()
            pltpu.make_async_copy(p_h.at[pl.ds(base, _B)],
                                  p_v.at[pl.ds(b * _B, _B)], sg[b]).wait()

        def compute(c, b):
            boff = b * _B

            def vb(j, c2):
                ia = boff + 2 * j + irow
                a = (plsc.load_gather(p_v, [ia, icol])
                     * plsc.load_gather(r_v, [ia, icol]))
                plsc.store_scatter(p_v, [ia, icol], a)
                return c2

            lax.fori_loop(0, _B // 2, vb, 0)

            def edge_body(kk, c2):
                for u in range(2):
                    row = boff + 2 * kk + u
                    sk = jnp.full((16,), row, jnp.int32)
                    for j in range(nj):
                        al = plsc.load_gather(p_v, [sk, jps[j]])
                        hv = h_v[row, pl.ds(16 * j, 16)]
                        m_v[row, pl.ds(16 * j, 16)] = hv * al
                return c2

            lax.fori_loop(0, _B // 2, edge_body, 0)

        def st_issue2(c, b, q):
            pltpu.sync_copy(m_v.at[pl.ds(b * _B, _B)],
                            acc_sh.at[dstr.at[q]], add=True)

        def st_wait2(c, b, q):
            pass

        def step(c, r, n1, n2, prev):
            if n1:
                idx_wait(c + 1, (r + 1) % 4)
            if n2:
                idx_issue(c + 2, (r + 2) % 4)
            gat_wait2(c, r % 2, r)
            if n1:
                gat_issue2(c + 1, (r + 1) % 2, (r + 1) % 4)
            compute(c, r % 2)
            if prev:
                st_wait2(c - 1, (r - 1) % 2, (r - 1) % 4)
            st_issue2(c, r % 2, r)

        idx_issue(0, 0)
        idx_wait(0, 0)
        gat_issue2(0, 0, 0)
        idx_issue(1, 1)
        step(0, 0, True, True, False)

        def quad(t, carry):
            for cc in range(4):
                step(4 * t + 1 + cc, (1 + cc) % 4, True, True, True)
            return carry

        lax.fori_loop(0, (nchunks - 5) // 4, quad, 0)
        for c in range(nchunks - 4, nchunks):
            step(c, c % 4, c + 1 < nchunks, c + 2 < nchunks, True)
        st_wait2(nchunks - 1, (nchunks - 1) % 2, (nchunks - 1) % 4)

        plsc.subcore_barrier()
        pltpu.sync_copy(acc_sh.at[pl.ds(r0, rpt)],
                        out_h.at[cid, pl.ds(r0, rpt)])

        @pl.when(sid == _NS - 1)
        def _():
            pltpu.sync_copy(acc_sh.at[pl.ds(_NS * rpt, tail)],
                            out_h.at[cid, pl.ds(_NS * rpt, tail)])

    return k(src_e, dst_e, table, p, rden, zF)


_BN = 1000


def _proj1(x, W1, As, Ad):
    N, Fin = x.shape
    Fo = W1.shape[1]
    H = As.shape[1]

    def body(x_r, w_r, as_r, ad_r, h_r, ts_r, td_r):
        h = jnp.dot(x_r[...], w_r[...], preferred_element_type=jnp.float32)
        h_r[...] = h
        ts_r[...] = jnp.dot(h, as_r[...], preferred_element_type=jnp.float32)
        td_r[...] = jnp.dot(h, ad_r[...], preferred_element_type=jnp.float32)

    return pl.pallas_call(
        body,
        grid=(N // _BN,),
        in_specs=[
            pl.BlockSpec((_BN, Fin), lambda i: (i, 0)),
            pl.BlockSpec((Fin, Fo), lambda i: (0, 0)),
            pl.BlockSpec((Fo, H), lambda i: (0, 0)),
            pl.BlockSpec((Fo, H), lambda i: (0, 0)),
        ],
        out_specs=[
            pl.BlockSpec((_BN, Fo), lambda i: (i, 0)),
            pl.BlockSpec((_BN, H), lambda i: (i, 0)),
            pl.BlockSpec((_BN, H), lambda i: (i, 0)),
        ],
        out_shape=[
            jax.ShapeDtypeStruct((N, Fo), jnp.float32),
            jax.ShapeDtypeStruct((N, H), jnp.float32),
            jax.ShapeDtypeStruct((N, H), jnp.float32),
        ],
    )(x, W1, As, Ad)


def _rden(den):
    T, N, H = den.shape

    def body(d_r, r_r):
        r_r[...] = 1.0 / (jnp.sum(d_r[...], axis=0) + 1e-16)

    return pl.pallas_call(
        body,
        grid=(N // _BN,),
        in_specs=[pl.BlockSpec((T, _BN, H), lambda i: (0, i, 0))],
        out_specs=pl.BlockSpec((_BN, H), lambda i: (i, 0)),
        out_shape=jax.ShapeDtypeStruct((N, H), jnp.float32),
    )(den)


def _layer2_prep(o, b1, W2, As2, Ad2):
    _, N, Fo = o.shape
    C = W2.shape[1]

    def body(o_r, b_r, w_r, as_r, ad_r, h2_r, ts_r, td_r):
        t = o_r[0] + o_r[1] + b_r[...]
        t = jnp.where(t > 0, t, jnp.exp(t) - 1.0)
        h2 = jnp.dot(t, w_r[...], preferred_element_type=jnp.float32)
        h2_r[...] = h2
        ts_r[...] = jnp.dot(h2, as_r[...], preferred_element_type=jnp.float32)
        td_r[...] = jnp.dot(h2, ad_r[...], preferred_element_type=jnp.float32)

    return pl.pallas_call(
        body,
        grid=(N // _BN,),
        in_specs=[
            pl.BlockSpec((2, _BN, Fo), lambda i: (0, i, 0)),
            pl.BlockSpec((1, Fo), lambda i: (0, 0)),
            pl.BlockSpec((Fo, C), lambda i: (0, 0)),
            pl.BlockSpec((C, 8), lambda i: (0, 0)),
            pl.BlockSpec((C, 8), lambda i: (0, 0)),
        ],
        out_specs=[
            pl.BlockSpec((_BN, C), lambda i: (i, 0)),
            pl.BlockSpec((_BN, 8), lambda i: (i, 0)),
            pl.BlockSpec((_BN, 8), lambda i: (i, 0)),
        ],
        out_shape=[
            jax.ShapeDtypeStruct((N, C), jnp.float32),
            jax.ShapeDtypeStruct((N, 8), jnp.float32),
            jax.ShapeDtypeStruct((N, 8), jnp.float32),
        ],
    )(o, b1, W2, As2, Ad2)


def _final(o, b2):
    _, N, C = o.shape

    def body(o_r, b_r, out_r):
        t = o_r[0] + o_r[1] + b_r[...]
        m = jnp.max(t, axis=1, keepdims=True)
        te = t - m
        lse = jnp.log(jnp.sum(jnp.exp(te), axis=1, keepdims=True))
        out_r[...] = te - lse

    return pl.pallas_call(
        body,
        grid=(N // _BN,),
        in_specs=[
            pl.BlockSpec((2, _BN, C), lambda i: (0, i, 0)),
            pl.BlockSpec((1, C), lambda i: (0, 0)),
        ],
        out_specs=pl.BlockSpec((_BN, C), lambda i: (i, 0)),
        out_shape=jax.ShapeDtypeStruct((N, C), jnp.float32),
    )(o, b2)


def kernel(x, edge_index, W1, a_src1, a_dst1, b1, W2, a_src2, a_dst2, b2):
    N, _ = x.shape
    H1, F1 = a_src1.shape
    C = W2.shape[1]

    eye = jnp.eye(H1, dtype=jnp.float32)
    As1 = (a_src1[:, :, None] * eye[:, None, :]).reshape(H1 * F1, H1)
    Ad1 = (a_dst1[:, :, None] * eye[:, None, :]).reshape(H1 * F1, H1)
    As2 = jnp.zeros((C, 8), jnp.float32).at[:, 0].set(a_src2[0])
    Ad2 = jnp.zeros((C, 8), jnp.float32).at[:, 0].set(a_dst2[0])

    z8 = jnp.zeros((N, 8), jnp.float32)
    z64 = jnp.zeros((N, H1 * F1), jnp.float32)
    z16 = jnp.zeros((N, C), jnp.float32)

    src_e = edge_index[0]
    dst_e = edge_index[1]

    h1, ts1, td1 = _proj1(x, W1, As1, Ad1)
    p1, den1 = _edge_den(src_e, dst_e, ts1, td1, z8)
    rden1 = _rden(den1)
    o1 = _edge_agg(src_e, dst_e, h1, p1, rden1, z64, Fh=F1)
    h2, ts2, td2 = _layer2_prep(o1, b1.reshape(1, -1), W2, As2, Ad2)
    p2, den2 = _edge_den(src_e, dst_e, ts2, td2, z8)
    rden2 = _rden(den2)
    o2 = _edge_agg(src_e, dst_e, h2, p2, rden2, z16, Fh=C)
    return _final(o2, b2.reshape(1, -1))
